# baseline (device time: 892250 ns/iter reference)
import jax
import jax.numpy as jnp
from jax import lax
from jax.experimental import pallas as pl
from jax.experimental.pallas import tpu as pltpu

N_DEV = 32


def kernel(x, w_mat):
    m, _ = x.shape
    _, n = w_mat.shape
    chunk = m // N_DEV
    half = n // 2
    n_steps = 2 * (N_DEV - 1)

    def body(x_ref, w_ref, out_ref, comm0, comm1,
             send0, recv0, send1, recv1, ack0, ack1):
        my = lax.axis_index("i")
        left = lax.rem(my + N_DEV - 1, N_DEV)
        right = lax.rem(my + 1, N_DEV)

        barrier_sem = pltpu.get_barrier_semaphore()
        for nbr in (left, right):
            pl.semaphore_signal(
                barrier_sem, inc=1,
                device_id=(nbr,), device_id_type=pl.DeviceIdType.MESH,
            )
        pl.semaphore_wait(barrier_sem, 2)

        out_ref[:, :] = jnp.dot(
            x_ref[:, :], w_ref[:, :], preferred_element_type=jnp.float32
        )

        def cs(c):
            return pl.ds(c * chunk, chunk)

        c0s = pl.ds(0, half)
        c1s = pl.ds(half, half)

        rdmas0 = [None, None]
        rdmas1 = [None, None]
        for s in range(n_steps):
            slot = s % 2
            if s < N_DEV - 1:
                sc0 = lax.rem(my - s + N_DEV, N_DEV)
                rc0 = lax.rem(my - s - 1 + N_DEV, N_DEV)
                sc1 = lax.rem(my + s, N_DEV)
                rc1 = lax.rem(my + s + 1, N_DEV)
            else:
                t = s - (N_DEV - 1)
                sc0 = lax.rem(my + 1 - t + N_DEV, N_DEV)
                rc0 = lax.rem(my - t + N_DEV, N_DEV)
                sc1 = lax.rem(my - 1 + t + N_DEV, N_DEV)
                rc1 = lax.rem(my + t, N_DEV)

            if s >= 2:
                pl.semaphore_wait(ack0, 1)
                pl.semaphore_wait(ack1, 1)
                rdmas0[slot].wait_send()
                rdmas1[slot].wait_send()

            rdmas0[slot] = pltpu.make_async_remote_copy(
                src_ref=out_ref.at[cs(sc0), c0s],
                dst_ref=comm0.at[slot],
                send_sem=send0.at[slot],
                recv_sem=recv0.at[slot],
                device_id=(right,),
                device_id_type=pl.DeviceIdType.MESH,
            )
            rdmas1[slot] = pltpu.make_async_remote_copy(
                src_ref=out_ref.at[cs(sc1), c1s],
                dst_ref=comm1.at[slot],
                send_sem=send1.at[slot],
                recv_sem=recv1.at[slot],
                device_id=(left,),
                device_id_type=pl.DeviceIdType.MESH,
            )
            rdmas0[slot].start()
            rdmas1[slot].start()
            rdmas0[slot].wait_recv()
            rdmas1[slot].wait_recv()

            if s < N_DEV - 1:
                out_ref[cs(rc0), c0s] += comm0[slot]
                out_ref[cs(rc1), c1s] += comm1[slot]
                if s == N_DEV - 2:
                    own0 = lax.rem(my + 1, N_DEV)
                    own1 = lax.rem(my + N_DEV - 1, N_DEV)
                    out_ref[cs(own0), c0s] = jnp.maximum(
                        out_ref[cs(own0), c0s], 0.0
                    )
                    out_ref[cs(own1), c1s] = jnp.maximum(
                        out_ref[cs(own1), c1s], 0.0
                    )
            else:
                out_ref[cs(rc0), c0s] = comm0[slot]
                out_ref[cs(rc1), c1s] = comm1[slot]

            if s < n_steps - 2:
                pl.semaphore_signal(
                    ack0, inc=1,
                    device_id=(left,), device_id_type=pl.DeviceIdType.MESH,
                )
                pl.semaphore_signal(
                    ack1, inc=1,
                    device_id=(right,), device_id_type=pl.DeviceIdType.MESH,
                )

        for slot in range(2):
            rdmas0[slot].wait_send()
            rdmas1[slot].wait_send()

    return pl.pallas_call(
        body,
        out_shape=jax.ShapeDtypeStruct((m, n), jnp.float32),
        in_specs=[
            pl.BlockSpec(memory_space=pltpu.VMEM),
            pl.BlockSpec(memory_space=pltpu.VMEM),
        ],
        out_specs=pl.BlockSpec(memory_space=pltpu.VMEM),
        scratch_shapes=[
            pltpu.VMEM((2, chunk, half), jnp.float32),
            pltpu.VMEM((2, chunk, half), jnp.float32),
            pltpu.SemaphoreType.DMA((2,)),
            pltpu.SemaphoreType.DMA((2,)),
            pltpu.SemaphoreType.DMA((2,)),
            pltpu.SemaphoreType.DMA((2,)),
            pltpu.SemaphoreType.REGULAR,
            pltpu.SemaphoreType.REGULAR,
        ],
        compiler_params=pltpu.CompilerParams(
            collective_id=0,
            vmem_limit_bytes=64 * 1024 * 1024,
        ),
    )(x, w_mat)


# device time: 761583 ns/iter; 1.1716x vs baseline; 1.1716x over previous
import jax
import jax.numpy as jnp
from jax import lax
from jax.experimental import pallas as pl
from jax.experimental.pallas import tpu as pltpu

N_DEV = 32
K_SLOTS = 4


def kernel(x, w_mat):
    m, _ = x.shape
    _, n = w_mat.shape
    chunk = m // N_DEV
    half = n // 2
    n_steps = 2 * (N_DEV - 1)

    def body(x_ref, w_ref, out_ref, comm0, comm1,
             send0, recv0, send1, recv1, ack0, ack1):
        my = lax.axis_index("i")
        left = lax.rem(my + N_DEV - 1, N_DEV)
        right = lax.rem(my + 1, N_DEV)

        barrier_sem = pltpu.get_barrier_semaphore()
        for nbr in (left, right):
            pl.semaphore_signal(
                barrier_sem, inc=1,
                device_id=(nbr,), device_id_type=pl.DeviceIdType.MESH,
            )
        pl.semaphore_wait(barrier_sem, 2)

        out_ref[:, :] = jnp.dot(
            x_ref[:, :], w_ref[:, :], preferred_element_type=jnp.float32
        )

        def cs(c):
            return pl.ds(c * chunk, chunk)

        c0s = pl.ds(0, half)
        c1s = pl.ds(half, half)

        def send_chunk(ring, s):
            if s < N_DEV - 1:
                return (
                    lax.rem(my - s + N_DEV, N_DEV) if ring == 0
                    else lax.rem(my + s, N_DEV)
                )
            t = s - (N_DEV - 1)
            return (
                lax.rem(my + 1 - t + N_DEV, N_DEV) if ring == 0
                else lax.rem(my - 1 + t + N_DEV, N_DEV)
            )

        def recv_chunk(ring, s):
            if s < N_DEV - 1:
                return (
                    lax.rem(my - s - 1 + N_DEV, N_DEV) if ring == 0
                    else lax.rem(my + s + 1, N_DEV)
                )
            t = s - (N_DEV - 1)
            return (
                lax.rem(my - t + N_DEV, N_DEV) if ring == 0
                else lax.rem(my + t, N_DEV)
            )

        comms = (comm0, comm1)
        sends = (send0, send1)
        recvs = (recv0, recv1)
        acks = (ack0, ack1)
        peers = ((right,), (left,))
        backs = ((left,), (right,))
        cols = (c0s, c1s)

        def make_rdma(ring, s):
            slot = s % K_SLOTS
            sc = send_chunk(ring, s)
            src = out_ref.at[cs(sc), cols[ring]]
            if s < N_DEV - 1:
                dst = comms[ring].at[slot]
            else:
                dst = out_ref.at[cs(sc), cols[ring]]
            return pltpu.make_async_remote_copy(
                src_ref=src,
                dst_ref=dst,
                send_sem=sends[ring].at[slot],
                recv_sem=recvs[ring].at[slot],
                device_id=peers[ring],
                device_id_type=pl.DeviceIdType.MESH,
            )

        rdmas = [[None] * K_SLOTS, [None] * K_SLOTS]

        for ring in (0, 1):
            rdmas[ring][0] = make_rdma(ring, 0)
            rdmas[ring][0].start()

        for s in range(n_steps):
            slot = s % K_SLOTS
            for ring in (0, 1):
                rdmas[ring][slot].wait_recv()
                if s < N_DEV - 1:
                    rc = recv_chunk(ring, s)
                    out_ref[cs(rc), cols[ring]] += comms[ring][slot]
                    if s == N_DEV - 2:
                        out_ref[cs(rc), cols[ring]] = jnp.maximum(
                            out_ref[cs(rc), cols[ring]], 0.0
                        )
                if s <= n_steps - 1 - K_SLOTS:
                    pl.semaphore_signal(
                        acks[ring], inc=1,
                        device_id=backs[ring],
                        device_id_type=pl.DeviceIdType.MESH,
                    )
                s2 = s + 1
                if s2 < n_steps:
                    slot2 = s2 % K_SLOTS
                    if s2 >= K_SLOTS:
                        pl.semaphore_wait(acks[ring], 1)
                        rdmas[ring][slot2].wait_send()
                    rdmas[ring][slot2] = make_rdma(ring, s2)
                    rdmas[ring][slot2].start()

        for ring in (0, 1):
            for sl in range(K_SLOTS):
                rdmas[ring][sl].wait_send()

    return pl.pallas_call(
        body,
        out_shape=jax.ShapeDtypeStruct((m, n), jnp.float32),
        in_specs=[
            pl.BlockSpec(memory_space=pltpu.VMEM),
            pl.BlockSpec(memory_space=pltpu.VMEM),
        ],
        out_specs=pl.BlockSpec(memory_space=pltpu.VMEM),
        scratch_shapes=[
            pltpu.VMEM((K_SLOTS, chunk, half), jnp.float32),
            pltpu.VMEM((K_SLOTS, chunk, half), jnp.float32),
            pltpu.SemaphoreType.DMA((K_SLOTS,)),
            pltpu.SemaphoreType.DMA((K_SLOTS,)),
            pltpu.SemaphoreType.DMA((K_SLOTS,)),
            pltpu.SemaphoreType.DMA((K_SLOTS,)),
            pltpu.SemaphoreType.REGULAR,
            pltpu.SemaphoreType.REGULAR,
        ],
        compiler_params=pltpu.CompilerParams(
            collective_id=0,
            vmem_limit_bytes=64 * 1024 * 1024,
        ),
    )(x, w_mat)
